# baseline (speedup 1.0000x reference)
import jax
import jax.numpy as jnp
from jax.experimental import pallas as pl
from jax.experimental.pallas import tpu as pltpu

_N_MAX = 20
_REG_WEIGHT = 0.01
_ROWS, _COLS = 128, 32768
_ROW_BLOCK = 32
_NSTEPS = _ROWS // _ROW_BLOCK
_K_PER_STEP = -(-_N_MAX // _NSTEPS)
_LAT_SUB = _COLS // 128


def _body(t_ref, p_ref, lat_ref, r_ref, total_ref, l1_ref, reg_ref,
          cur_ref, col_ref, row_ref):
    step = pl.program_id(0)

    bsum = jnp.sum(jnp.abs(t_ref[...] - p_ref[...]))

    @pl.when(step == 0)
    def _init():
        l1_ref[...] = jnp.reshape(bsum, (1, 1))
        cur_ref[...] = lat_ref[...]
        col_ref[...] = jnp.zeros_like(col_ref)
        row_ref[...] = jnp.zeros_like(row_ref)

    @pl.when(step != 0)
    def _acc():
        l1_ref[...] += jnp.reshape(bsum, (1, 1))

    gidx = (jax.lax.broadcasted_iota(jnp.int32, (_LAT_SUB, 128), 0) * 128
            + jax.lax.broadcasted_iota(jnp.int32, (_LAT_SUB, 128), 1))
    sub = jax.lax.broadcasted_iota(jnp.int32, (32, 128), 0)
    lane = jax.lax.broadcasted_iota(jnp.int32, (32, 128), 1)
    rx = r_ref[0]
    ry = r_ref[1]
    rz = r_ref[2]

    for j in range(_K_PER_STEP):
        k = step * _K_PER_STEP + j

        @pl.when(k < _N_MAX)
        def _round():
            cur = cur_ref[...]
            idx = jnp.argmax(cur).astype(jnp.int32)
            pickb = gidx == idx
            pick = pickb.astype(jnp.float32)
            xk = jnp.sum(rx * pick)
            yk = jnp.sum(ry * pick)
            zk = jnp.sum(rz * pick)
            cur_ref[...] = jnp.where(pickb, jnp.float32(-3.4e38), cur)
            rmask = (sub == k).astype(jnp.float32)
            cmask = (lane == k).astype(jnp.float32)
            col_ref[0] += xk * rmask
            col_ref[1] += yk * rmask
            col_ref[2] += zk * rmask
            row_ref[0] += xk * cmask
            row_ref[1] += yk * cmask
            row_ref[2] += zk * cmask

    @pl.when(step == _NSTEPS - 1)
    def _fin():
        dx = col_ref[0] - row_ref[0]
        dy = col_ref[1] - row_ref[1]
        dz = col_ref[2] - row_ref[2]
        dist = jnp.sqrt(dx * dx + dy * dy + dz * dz)
        pairmask = ((sub < lane) & (lane < _N_MAX)).astype(jnp.float32)
        npairs = float(_N_MAX * (_N_MAX - 1) // 2)
        mean = jnp.sum(dist * pairmask) / npairs
        var = jnp.sum((dist - mean) ** 2 * pairmask) / (npairs - 1.0)
        regw = jnp.reshape(_REG_WEIGHT * jnp.sqrt(var), (1, 1))
        reg_ref[...] = regw
        l1 = l1_ref[...] / float(_ROWS * _COLS)
        l1_ref[...] = l1
        total_ref[...] = l1 + regw


def kernel(target, pred, latent, R_xyz):
    lat2d = latent.reshape(_LAT_SUB, 128)
    r3d = R_xyz.reshape(3, _LAT_SUB, 128)
    out = pl.pallas_call(
        _body,
        grid=(_NSTEPS,),
        in_specs=[
            pl.BlockSpec((_ROW_BLOCK, _COLS), lambda i: (i, 0)),
            pl.BlockSpec((_ROW_BLOCK, _COLS), lambda i: (i, 0)),
            pl.BlockSpec((_LAT_SUB, 128), lambda i: (0, 0)),
            pl.BlockSpec((3, _LAT_SUB, 128), lambda i: (0, 0, 0)),
        ],
        out_specs=[
            pl.BlockSpec((1, 1), lambda i: (0, 0)),
            pl.BlockSpec((1, 1), lambda i: (0, 0)),
            pl.BlockSpec((1, 1), lambda i: (0, 0)),
        ],
        out_shape=[
            jax.ShapeDtypeStruct((1, 1), jnp.float32),
            jax.ShapeDtypeStruct((1, 1), jnp.float32),
            jax.ShapeDtypeStruct((1, 1), jnp.float32),
        ],
        scratch_shapes=[
            pltpu.VMEM((_LAT_SUB, 128), jnp.float32),
            pltpu.VMEM((3, 32, 128), jnp.float32),
            pltpu.VMEM((3, 32, 128), jnp.float32),
        ],
        compiler_params=pltpu.CompilerParams(
            dimension_semantics=("arbitrary",),
        ),
    )(target, pred, lat2d, r3d)
    total, l1, reg = out
    return (total[0, 0], l1[0, 0], reg[0, 0])


# speedup vs baseline: 1.1125x; 1.1125x over previous
import jax
import jax.numpy as jnp
from jax.experimental import pallas as pl
from jax.experimental.pallas import tpu as pltpu

_N_MAX = 20
_REG_WEIGHT = 0.01
_ROWS, _COLS = 128, 32768
_ROW_BLOCK = 32
_NSTEPS = _ROWS // _ROW_BLOCK
_K_PER_STEP = -(-_N_MAX // _NSTEPS)
_LAT_SUB = _COLS // 128


def _body(t_ref, p_ref, lat_ref, r_ref, total_ref, l1_ref, reg_ref,
          cur_ref, col_ref, row_ref):
    step = pl.program_id(0)

    bsum = jnp.sum(jnp.abs(t_ref[...] - p_ref[...]))

    @pl.when(step == 0)
    def _init():
        l1_ref[...] = jnp.reshape(bsum, (1, 1))
        cur_ref[...] = lat_ref[...]
        col_ref[...] = jnp.zeros_like(col_ref)
        row_ref[...] = jnp.zeros_like(row_ref)

    @pl.when(step != 0)
    def _acc():
        l1_ref[...] += jnp.reshape(bsum, (1, 1))

    gidx = (jax.lax.broadcasted_iota(jnp.int32, (_LAT_SUB, 128), 0) * 128
            + jax.lax.broadcasted_iota(jnp.int32, (_LAT_SUB, 128), 1))
    sub = jax.lax.broadcasted_iota(jnp.int32, (32, 128), 0)
    lane = jax.lax.broadcasted_iota(jnp.int32, (32, 128), 1)
    rx = r_ref[0]
    ry = r_ref[1]
    rz = r_ref[2]

    for j in range(_K_PER_STEP):
        k = step * _K_PER_STEP + j

        @pl.when(k < _N_MAX)
        def _round():
            cur = cur_ref[...]
            m = jnp.max(cur)
            idx = jnp.min(jnp.where(cur == m, gidx, jnp.int32(2**30)))
            pickb = gidx == idx
            pick = pickb.astype(jnp.float32)
            xk = jnp.sum(rx * pick)
            yk = jnp.sum(ry * pick)
            zk = jnp.sum(rz * pick)
            cur_ref[...] = jnp.where(pickb, jnp.float32(-3.4e38), cur)
            rmask = (sub == k).astype(jnp.float32)
            cmask = (lane == k).astype(jnp.float32)
            col_ref[0] += xk * rmask
            col_ref[1] += yk * rmask
            col_ref[2] += zk * rmask
            row_ref[0] += xk * cmask
            row_ref[1] += yk * cmask
            row_ref[2] += zk * cmask

    @pl.when(step == _NSTEPS - 1)
    def _fin():
        dx = col_ref[0] - row_ref[0]
        dy = col_ref[1] - row_ref[1]
        dz = col_ref[2] - row_ref[2]
        dist = jnp.sqrt(dx * dx + dy * dy + dz * dz)
        pairmask = ((sub < lane) & (lane < _N_MAX)).astype(jnp.float32)
        npairs = float(_N_MAX * (_N_MAX - 1) // 2)
        mean = jnp.sum(dist * pairmask) / npairs
        var = jnp.sum((dist - mean) ** 2 * pairmask) / (npairs - 1.0)
        regw = jnp.reshape(_REG_WEIGHT * jnp.sqrt(var), (1, 1))
        reg_ref[...] = regw
        l1 = l1_ref[...] / float(_ROWS * _COLS)
        l1_ref[...] = l1
        total_ref[...] = l1 + regw


def kernel(target, pred, latent, R_xyz):
    lat2d = latent.reshape(_LAT_SUB, 128)
    r3d = R_xyz.reshape(3, _LAT_SUB, 128)
    out = pl.pallas_call(
        _body,
        grid=(_NSTEPS,),
        in_specs=[
            pl.BlockSpec((_ROW_BLOCK, _COLS), lambda i: (i, 0)),
            pl.BlockSpec((_ROW_BLOCK, _COLS), lambda i: (i, 0)),
            pl.BlockSpec((_LAT_SUB, 128), lambda i: (0, 0)),
            pl.BlockSpec((3, _LAT_SUB, 128), lambda i: (0, 0, 0)),
        ],
        out_specs=[
            pl.BlockSpec((1, 1), lambda i: (0, 0)),
            pl.BlockSpec((1, 1), lambda i: (0, 0)),
            pl.BlockSpec((1, 1), lambda i: (0, 0)),
        ],
        out_shape=[
            jax.ShapeDtypeStruct((1, 1), jnp.float32),
            jax.ShapeDtypeStruct((1, 1), jnp.float32),
            jax.ShapeDtypeStruct((1, 1), jnp.float32),
        ],
        scratch_shapes=[
            pltpu.VMEM((_LAT_SUB, 128), jnp.float32),
            pltpu.VMEM((3, 32, 128), jnp.float32),
            pltpu.VMEM((3, 32, 128), jnp.float32),
        ],
        compiler_params=pltpu.CompilerParams(
            dimension_semantics=("arbitrary",),
        ),
    )(target, pred, lat2d, r3d)
    total, l1, reg = out
    return (total[0, 0], l1[0, 0], reg[0, 0])
